# baseline (speedup 1.0000x reference)
import jax
import jax.numpy as jnp
from jax import lax
from jax.experimental import pallas as pl
from jax.experimental.pallas import tpu as pltpu
from jax.experimental.pallas import tpu_sc as plsc

_DIM = 32
_NC, _NS = 2, 16
_NW = _NC * _NS
_C = 512
_L = 200
_B = 4096
_VP = 1000064
_KC = 512
_NBLK = (_VP + _KC - 1) // _KC
_BPW = (_NBLK + _NW - 1) // _NW


def _transpose_body(tblT_hbm, out_hbm, in_v, out_v, semi0, semi1, semo0,
                    semo1):
    wid = lax.axis_index("s") * _NC + lax.axis_index("c")
    lim = jnp.minimum(_NBLK, (wid + 1) * _BPW)
    semi = (semi0, semi1)
    semo = (semo0, semo1)

    iota16 = lax.iota(jnp.int32, 16)
    rowc = []
    colc = []
    for g in range(8):
        w = g * 16 + iota16
        rowc.append(lax.bitwise_and(w, 31))
        colc.append(lax.shift_right_logical(w, 5))

    def blk(bi):
        return wid * _BPW + bi

    def col0(bi):
        return pl.multiple_of(jnp.minimum(blk(bi) * _KC, _VP - _KC), 128)

    def in_copy(bi, s):
        return pltpu.make_async_copy(
            tblT_hbm.at[:, pl.ds(col0(bi), _KC)], in_v.at[s], semi[s])

    def out_copy(bi, s):
        return pltpu.make_async_copy(
            out_v.at[s],
            out_hbm.at[pl.ds(pl.multiple_of(col0(bi) // 4, 32), _KC // 4)],
            semo[s])

    def transpose(s):
        @pl.loop(0, _KC // 4, unroll=2)
        def _fat(f):
            colb = jnp.broadcast_to(4 * f, (16,))
            for g in range(8):
                vec = plsc.load_gather(in_v.at[s], [rowc[g], colb + colc[g]])
                out_v[s, f, pl.ds(g * 16, 16)] = vec

    @pl.when(blk(0) < lim)
    def _():
        in_copy(0, 0).start()

    @pl.when(blk(1) < lim)
    def _():
        in_copy(1, 1).start()

    @pl.loop(0, (_BPW + 2) // 2)
    def _pair(t):
        for b in range(2):
            s = b
            bi = 2 * t + b

            @pl.when(blk(bi) < lim)
            def _():
                in_copy(bi, s).wait()

                @pl.when(bi >= 2)
                def _():
                    out_copy(bi - 2, s).wait()

                transpose(s)

                @pl.when(blk(bi + 2) < lim)
                def _():
                    in_copy(bi + 2, s).start()

                out_copy(bi, s).start()

    n_valid = lim - wid * _BPW
    for s in range(2):
        last = ((n_valid - 1 - s) // 2) * 2 + s

        @pl.when(n_valid > s)
        def _():
            out_copy(last, s).wait()


def _gather_body(idx_hbm, table_hbm, out_hbm, idx_v, rows_v, out_t,
                 semi0, semi1, semg0, semg1, semo0, semo1):
    n_rows = idx_hbm.shape[0]
    r_per_w = n_rows // _NW
    n_chunks = r_per_w // _C
    wid = lax.axis_index("s") * _NC + lax.axis_index("c")
    base = wid * r_per_w

    semi = (semi0, semi1)
    semg = (semg0, semg1)
    semo = (semo0, semo1)

    iota16 = lax.iota(jnp.int32, 16)

    def idx_copy(c, s):
        return pltpu.make_async_copy(
            idx_hbm.at[pl.ds(base + c * _C, _C)], idx_v.at[s], semi[s])

    def fire_gather(s):
        pltpu.async_copy(table_hbm.at[idx_v.at[s]], rows_v.at[s], semg[s])

    def drain_gather(s):
        pltpu.make_async_copy(table_hbm.at[pl.ds(0, _C)], rows_v.at[s],
                              semg[s]).wait()

    def out_pieces(c, s):
        flat0 = base + c * _C
        l = flat0 // _B
        j0 = (flat0 % _B) // 128
        return [pltpu.make_async_copy(
                    out_t.at[s, pl.ds(i * 4096, 4096)],
                    out_hbm.at[l, pl.ds(i * 32768 + j0 * 1024, 4096)],
                    semo[s])
                for i in range(4)]

    def out_start(c, s):
        for p in out_pieces(c, s):
            p.start()

    def out_wait(c, s):
        for p in out_pieces(c, s):
            p.wait()

    def transpose(s):
        @pl.loop(0, 128, unroll=2)
        def _m(m):
            i = lax.shift_right_logical(m, 5)
            jj = lax.bitwise_and(lax.shift_right_logical(m, 3), 3)
            s8 = lax.bitwise_and(m, 7)
            col = jnp.broadcast_to(i * 8 + s8, (16,))
            rowv = jnp.broadcast_to(jj * 128, (16,)) + iota16
            obase = m * 128
            for g in range(8):
                vec = plsc.load_gather(rows_v.at[s], [rowv + g * 16, col])
                out_t[s, pl.ds(obase + g * 16, 16)] = vec

    idx_copy(0, 0).start()
    idx_copy(1, 1).start()
    idx_copy(0, 0).wait()
    fire_gather(0)

    @pl.loop(0, n_chunks // 2)
    def _pair(t):
        for b in range(2):
            c = 2 * t + b
            s = b
            drain_gather(s)

            @pl.when(c + 2 < n_chunks)
            def _():
                idx_copy(c + 2, s).start()

            @pl.when(c + 1 < n_chunks)
            def _():
                idx_copy(c + 1, 1 - s).wait()
                fire_gather(1 - s)

            @pl.when(c >= 2)
            def _():
                out_wait(c - 2, s)

            transpose(s)
            out_start(c, s)

    out_wait(n_chunks - 2, 0)
    out_wait(n_chunks - 1, 1)


def kernel(inp, table):
    b, l = inp.shape
    n = b * l
    idx = inp.T.reshape(n).astype(jnp.int32)
    mesh = plsc.VectorSubcoreMesh(core_axis_name="c", subcore_axis_name="s")

    tbl_rows = pl.kernel(
        _transpose_body,
        out_type=jax.ShapeDtypeStruct((_VP // 4, 128), table.dtype),
        mesh=mesh,
        scratch_types=[
            pltpu.VMEM((2, _DIM, _KC), jnp.float32),
            pltpu.VMEM((2, _KC // 4, 128), jnp.float32),
            pltpu.SemaphoreType.DMA,
            pltpu.SemaphoreType.DMA,
            pltpu.SemaphoreType.DMA,
            pltpu.SemaphoreType.DMA,
        ],
        compiler_params=pltpu.CompilerParams(use_tc_tiling_on_sc=True,
                                             needs_layout_passes=False),
    )(table.T)

    out2 = pl.kernel(
        _gather_body,
        out_type=jax.ShapeDtypeStruct((_L, 131072), table.dtype),
        mesh=mesh,
        scratch_types=[
            pltpu.VMEM((2, _C), jnp.int32),
            pltpu.VMEM((2, _C, _DIM), jnp.float32),
            pltpu.VMEM((2, 16384), jnp.float32),
            pltpu.SemaphoreType.DMA,
            pltpu.SemaphoreType.DMA,
            pltpu.SemaphoreType.DMA,
            pltpu.SemaphoreType.DMA,
            pltpu.SemaphoreType.DMA,
            pltpu.SemaphoreType.DMA,
        ],
        compiler_params=pltpu.CompilerParams(use_tc_tiling_on_sc=False,
                                             needs_layout_passes=False),
    )(idx, tbl_rows.reshape(_VP, _DIM))

    return (out2.reshape(_L, 4, 32, 8, 128)
                .transpose(2, 4, 0, 1, 3).reshape(b, l, _DIM))


# speedup vs baseline: 1.3862x; 1.3862x over previous
import jax
import jax.numpy as jnp
from jax import lax
from jax.experimental import pallas as pl
from jax.experimental.pallas import tpu as pltpu
from jax.experimental.pallas import tpu_sc as plsc

_DIM = 32
_NC, _NS = 2, 16
_NW = _NC * _NS
_C = 512
_L = 200
_B = 4096
_VP = 1000064
_KC = 512
_NBLK = (_VP + _KC - 1) // _KC
_BPW = (_NBLK + _NW - 1) // _NW


def _transpose_body(tblT_hbm, out_hbm, in_v, out_v, semi0, semi1, semo0,
                    semo1):
    wid = lax.axis_index("s") * _NC + lax.axis_index("c")
    lim = jnp.minimum(_NBLK, (wid + 1) * _BPW)
    semi = (semi0, semi1)
    semo = (semo0, semo1)

    iota16 = lax.iota(jnp.int32, 16)
    rowc = []
    colc = []
    for g in range(8):
        w = g * 16 + iota16
        rowc.append(lax.bitwise_and(w, 31))
        colc.append(lax.shift_right_logical(w, 5))

    def blk(bi):
        return wid * _BPW + bi

    def col0(bi):
        return pl.multiple_of(jnp.minimum(blk(bi) * _KC, _VP - _KC), 128)

    def in_copy(bi, s):
        return pltpu.make_async_copy(
            tblT_hbm.at[:, pl.ds(col0(bi), _KC)],
            in_v.at[s, :, pl.ds(0, _KC)], semi[s])

    def out_copy(bi, s):
        return pltpu.make_async_copy(
            out_v.at[s],
            out_hbm.at[pl.ds(pl.multiple_of(col0(bi) // 4, 32), _KC // 4)],
            semo[s])

    def transpose(s):
        @pl.loop(0, _KC // 4, unroll=2)
        def _fat(f):
            colb = jnp.broadcast_to(4 * f, (16,))
            for g in range(8):
                vec = plsc.load_gather(in_v.at[s], [rowc[g], colb + colc[g]])
                out_v[s, f, pl.ds(g * 16, 16)] = vec

    @pl.when(blk(0) < lim)
    def _():
        in_copy(0, 0).start()

    @pl.when(blk(1) < lim)
    def _():
        in_copy(1, 1).start()

    @pl.loop(0, (_BPW + 2) // 2)
    def _pair(t):
        for b in range(2):
            s = b
            bi = 2 * t + b

            @pl.when(blk(bi) < lim)
            def _():
                in_copy(bi, s).wait()

                @pl.when(bi >= 2)
                def _():
                    out_copy(bi - 2, s).wait()

                transpose(s)

                @pl.when(blk(bi + 2) < lim)
                def _():
                    in_copy(bi + 2, s).start()

                out_copy(bi, s).start()

    n_valid = lim - wid * _BPW
    for s in range(2):
        last = ((n_valid - 1 - s) // 2) * 2 + s

        @pl.when(n_valid > s)
        def _():
            out_copy(last, s).wait()


def _gather_body(idx_hbm, table_hbm, out_hbm, idx_v, rows_v, out_t,
                 semi0, semi1, semg0, semg1, semo0, semo1):
    n_rows = idx_hbm.shape[0]
    r_per_w = n_rows // _NW
    n_chunks = r_per_w // _C
    wid = lax.axis_index("s") * _NC + lax.axis_index("c")
    base = wid * r_per_w

    semi = (semi0, semi1)
    semg = (semg0, semg1)
    semo = (semo0, semo1)

    iota16 = lax.iota(jnp.int32, 16)
    mconst = []
    for h in range(2):
        d = h * 16 + iota16
        mconst.append(lax.shift_right_logical(d, 3) * 32 +
                      lax.bitwise_and(d, 7))

    def idx_copy(c, s):
        return pltpu.make_async_copy(
            idx_hbm.at[pl.ds(base + c * _C, _C)], idx_v.at[s], semi[s])

    def fire_gather(s):
        pltpu.async_copy(table_hbm.at[idx_v.at[s]], rows_v.at[s], semg[s])

    def drain_gather(s):
        pltpu.make_async_copy(table_hbm.at[pl.ds(0, _C)], rows_v.at[s],
                              semg[s]).wait()

    def out_pieces(c, s):
        flat0 = base + c * _C
        l = flat0 // _B
        j0 = (flat0 % _B) // 128
        return [pltpu.make_async_copy(
                    out_t.at[s, pl.ds(i * 32, 32), pl.ds(0, 128)],
                    out_hbm.at[l, i, pl.ds(j0 * 8, 32), :],
                    semo[s])
                for i in range(4)]

    def out_start(c, s):
        for p in out_pieces(c, s):
            p.start()

    def out_wait(c, s):
        for p in out_pieces(c, s):
            p.wait()

    def transpose(s):
        @pl.loop(0, _C, unroll=2)
        def _row(r):
            mrow = jnp.broadcast_to(
                lax.bitwise_and(lax.shift_right_logical(r, 7), 3) * 8, (16,))
            lane = jnp.broadcast_to(lax.bitwise_and(r, 127), (16,))
            for h in range(2):
                vec = rows_v[s, r, pl.ds(h * 16, 16)]
                plsc.store_scatter(out_t.at[s], [mrow + mconst[h], lane], vec)

    idx_copy(0, 0).start()
    idx_copy(1, 1).start()
    idx_copy(0, 0).wait()
    fire_gather(0)

    @pl.loop(0, n_chunks // 2)
    def _pair(t):
        for b in range(2):
            c = 2 * t + b
            s = b
            drain_gather(s)

            @pl.when(c + 2 < n_chunks)
            def _():
                idx_copy(c + 2, s).start()

            @pl.when(c + 1 < n_chunks)
            def _():
                idx_copy(c + 1, 1 - s).wait()
                fire_gather(1 - s)

            @pl.when(c >= 2)
            def _():
                out_wait(c - 2, s)

            transpose(s)
            out_start(c, s)

    out_wait(n_chunks - 2, 0)
    out_wait(n_chunks - 1, 1)


def kernel(inp, table):
    b, l = inp.shape
    n = b * l
    idx = inp.T.reshape(n).astype(jnp.int32)
    mesh = plsc.VectorSubcoreMesh(core_axis_name="c", subcore_axis_name="s")

    tbl_rows = pl.kernel(
        _transpose_body,
        out_type=jax.ShapeDtypeStruct((_VP // 4, 128), table.dtype),
        mesh=mesh,
        scratch_types=[
            pltpu.VMEM((2, _DIM, _KC + 1), jnp.float32),
            pltpu.VMEM((2, _KC // 4, 128), jnp.float32),
            pltpu.SemaphoreType.DMA,
            pltpu.SemaphoreType.DMA,
            pltpu.SemaphoreType.DMA,
            pltpu.SemaphoreType.DMA,
        ],
        compiler_params=pltpu.CompilerParams(use_tc_tiling_on_sc=True,
                                             needs_layout_passes=False),
    )(table.T)

    out2 = pl.kernel(
        _gather_body,
        out_type=jax.ShapeDtypeStruct((_L, 4, 256, 128), table.dtype),
        mesh=mesh,
        scratch_types=[
            pltpu.VMEM((2, _C), jnp.int32),
            pltpu.VMEM((2, _C, _DIM), jnp.float32),
            pltpu.VMEM((2, 128, 129), jnp.float32),
            pltpu.SemaphoreType.DMA,
            pltpu.SemaphoreType.DMA,
            pltpu.SemaphoreType.DMA,
            pltpu.SemaphoreType.DMA,
            pltpu.SemaphoreType.DMA,
            pltpu.SemaphoreType.DMA,
        ],
        compiler_params=pltpu.CompilerParams(use_tc_tiling_on_sc=False,
                                             needs_layout_passes=False),
    )(idx, tbl_rows.reshape(_VP, _DIM))

    return (out2.reshape(_L, 4, 32, 8, 128)
                .transpose(2, 4, 0, 1, 3).reshape(b, l, _DIM))


# speedup vs baseline: 1.3879x; 1.0013x over previous
import jax
import jax.numpy as jnp
from jax import lax
from jax.experimental import pallas as pl
from jax.experimental.pallas import tpu as pltpu
from jax.experimental.pallas import tpu_sc as plsc

_DIM = 32
_NC, _NS = 2, 16
_NW = _NC * _NS
_C = 512
_L = 200
_B = 4096
_VP = 1000064
_KC = 512
_NBLK = (_VP + _KC - 1) // _KC
_BPW = (_NBLK + _NW - 1) // _NW


def _transpose_body(tblT_hbm, out_hbm, in_v, out_v, semi0, semi1, semo0,
                    semo1):
    wid = lax.axis_index("s") * _NC + lax.axis_index("c")
    lim = jnp.minimum(_NBLK, (wid + 1) * _BPW)
    semi = (semi0, semi1)
    semo = (semo0, semo1)

    iota16 = lax.iota(jnp.int32, 16)
    rowc = []
    colc = []
    for g in range(8):
        w = g * 16 + iota16
        rowc.append(lax.bitwise_and(w, 31))
        colc.append(lax.shift_right_logical(w, 5))

    def blk(bi):
        return wid * _BPW + bi

    def col0(bi):
        return pl.multiple_of(jnp.minimum(blk(bi) * _KC, _VP - _KC), 128)

    def in_pieces(bi, s):
        return [pltpu.make_async_copy(
                    tblT_hbm.at[pl.ds(i * 8, 8), pl.ds(col0(bi), _KC)],
                    in_v.at[s, pl.ds(i * 8, 8), pl.ds(0, _KC)], semi[s])
                for i in range(4)]

    def out_copy(bi, s):
        return pltpu.make_async_copy(
            out_v.at[s],
            out_hbm.at[pl.ds(pl.multiple_of(col0(bi) // 4, 32), _KC // 4)],
            semo[s])

    def transpose(s):
        @pl.loop(0, _KC // 4, unroll=2)
        def _fat(f):
            colb = jnp.broadcast_to(4 * f, (16,))
            for g in range(8):
                vec = plsc.load_gather(in_v.at[s], [rowc[g], colb + colc[g]])
                out_v[s, f, pl.ds(g * 16, 16)] = vec

    @pl.when(blk(0) < lim)
    def _():
        for p in in_pieces(0, 0):
            p.start()

    @pl.when(blk(1) < lim)
    def _():
        for p in in_pieces(1, 1):
            p.start()

    @pl.loop(0, (_BPW + 2) // 2)
    def _pair(t):
        for b in range(2):
            s = b
            bi = 2 * t + b

            @pl.when(blk(bi) < lim)
            def _():
                for p in in_pieces(bi, s):
                    p.wait()

                @pl.when(bi >= 2)
                def _():
                    out_copy(bi - 2, s).wait()

                transpose(s)

                @pl.when(blk(bi + 2) < lim)
                def _():
                    for p in in_pieces(bi + 2, s):
                        p.start()

                out_copy(bi, s).start()

    n_valid = lim - wid * _BPW
    for s in range(2):
        last = ((n_valid - 1 - s) // 2) * 2 + s

        @pl.when(n_valid > s)
        def _():
            out_copy(last, s).wait()


def _gather_body(idx_hbm, table_hbm, out_hbm, idx_v, rows_v, out_t,
                 semi0, semi1, semg0, semg1, semo0, semo1):
    n_rows = idx_hbm.shape[0]
    r_per_w = n_rows // _NW
    n_chunks = r_per_w // _C
    wid = lax.axis_index("s") * _NC + lax.axis_index("c")
    base = wid * r_per_w

    semi = (semi0, semi1)
    semg = (semg0, semg1)
    semo = (semo0, semo1)

    iota16 = lax.iota(jnp.int32, 16)
    mconst = []
    for h in range(2):
        d = h * 16 + iota16
        mconst.append(lax.shift_right_logical(d, 3) * 32 +
                      lax.bitwise_and(d, 7))

    def idx_copy(c, s):
        return pltpu.make_async_copy(
            idx_hbm.at[pl.ds(base + c * _C, _C)], idx_v.at[s], semi[s])

    def fire_gather(s):
        pltpu.async_copy(table_hbm.at[idx_v.at[s]], rows_v.at[s], semg[s])

    def drain_gather(s):
        pltpu.make_async_copy(table_hbm.at[pl.ds(0, _C)], rows_v.at[s],
                              semg[s]).wait()

    def out_pieces(c, s):
        flat0 = base + c * _C
        l = flat0 // _B
        j0 = (flat0 % _B) // 128
        return [pltpu.make_async_copy(
                    out_t.at[s, pl.ds(i * 32, 32), pl.ds(0, 128)],
                    out_hbm.at[l, i, pl.ds(j0 * 8, 32), :],
                    semo[s])
                for i in range(4)]

    def out_start(c, s):
        for p in out_pieces(c, s):
            p.start()

    def out_wait(c, s):
        for p in out_pieces(c, s):
            p.wait()

    def transpose(s):
        @pl.loop(0, _C, unroll=2)
        def _row(r):
            mrow = jnp.broadcast_to(
                lax.bitwise_and(lax.shift_right_logical(r, 7), 3) * 8, (16,))
            lane = jnp.broadcast_to(lax.bitwise_and(r, 127), (16,))
            for h in range(2):
                vec = rows_v[s, r, pl.ds(h * 16, 16)]
                plsc.store_scatter(out_t.at[s], [mrow + mconst[h], lane], vec)

    idx_copy(0, 0).start()
    idx_copy(1, 1).start()
    idx_copy(0, 0).wait()
    fire_gather(0)

    @pl.loop(0, n_chunks // 2)
    def _pair(t):
        for b in range(2):
            c = 2 * t + b
            s = b
            drain_gather(s)

            @pl.when(c + 2 < n_chunks)
            def _():
                idx_copy(c + 2, s).start()

            @pl.when(c + 1 < n_chunks)
            def _():
                idx_copy(c + 1, 1 - s).wait()
                fire_gather(1 - s)

            @pl.when(c >= 2)
            def _():
                out_wait(c - 2, s)

            transpose(s)
            out_start(c, s)

    out_wait(n_chunks - 2, 0)
    out_wait(n_chunks - 1, 1)


def kernel(inp, table):
    b, l = inp.shape
    n = b * l
    idx = inp.T.reshape(n).astype(jnp.int32)
    mesh = plsc.VectorSubcoreMesh(core_axis_name="c", subcore_axis_name="s")

    tbl_rows = pl.kernel(
        _transpose_body,
        out_type=jax.ShapeDtypeStruct((_VP // 4, 128), table.dtype),
        mesh=mesh,
        scratch_types=[
            pltpu.VMEM((2, _DIM, _KC + 1), jnp.float32),
            pltpu.VMEM((2, _KC // 4, 128), jnp.float32),
            pltpu.SemaphoreType.DMA,
            pltpu.SemaphoreType.DMA,
            pltpu.SemaphoreType.DMA,
            pltpu.SemaphoreType.DMA,
        ],
        compiler_params=pltpu.CompilerParams(use_tc_tiling_on_sc=True,
                                             needs_layout_passes=False),
    )(table.T)

    out2 = pl.kernel(
        _gather_body,
        out_type=jax.ShapeDtypeStruct((_L, 4, 256, 128), table.dtype),
        mesh=mesh,
        scratch_types=[
            pltpu.VMEM((2, _C), jnp.int32),
            pltpu.VMEM((2, _C, _DIM), jnp.float32),
            pltpu.VMEM((2, 128, 129), jnp.float32),
            pltpu.SemaphoreType.DMA,
            pltpu.SemaphoreType.DMA,
            pltpu.SemaphoreType.DMA,
            pltpu.SemaphoreType.DMA,
            pltpu.SemaphoreType.DMA,
            pltpu.SemaphoreType.DMA,
        ],
        compiler_params=pltpu.CompilerParams(use_tc_tiling_on_sc=False,
                                             needs_layout_passes=False),
    )(idx, tbl_rows.reshape(_VP, _DIM))

    return (out2.reshape(_L, 4, 32, 8, 128)
                .transpose(2, 4, 0, 1, 3).reshape(b, l, _DIM))


# speedup vs baseline: 2.5708x; 1.8523x over previous
import jax
import jax.numpy as jnp
from jax import lax
from jax.experimental import pallas as pl
from jax.experimental.pallas import tpu as pltpu
from jax.experimental.pallas import tpu_sc as plsc

_DIM = 32
_NC, _NS = 2, 16
_NW = _NC * _NS
_C = 512
_L = 200
_B = 4096
_VP = 1000064
_KC = 512
_NBLK = (_VP + _KC - 1) // _KC
_BPW = (_NBLK + _NW - 1) // _NW


def _transpose_body(tblT_hbm, out_hbm, in_v, out_v, semi0, semi1, semo0,
                    semo1):
    wid = lax.axis_index("s") * _NC + lax.axis_index("c")
    lim = jnp.minimum(_NBLK, (wid + 1) * _BPW)
    semi = (semi0, semi1)
    semo = (semo0, semo1)

    iota16 = lax.iota(jnp.int32, 16)
    wvec = [lax.bitwise_and(d + iota16, 15) for d in range(16)]
    fvec = [lax.shift_right_logical(w, 2) for w in wvec]
    ovec = [lax.bitwise_and(w, 3) * 32 + iota16 for w in wvec]

    def blk(bi):
        return wid * _BPW + bi

    def col0(bi):
        return pl.multiple_of(jnp.minimum(blk(bi) * _KC, _VP - _KC), 128)

    def in_pieces(bi, s):
        return [pltpu.make_async_copy(
                    tblT_hbm.at[pl.ds(i * 8, 8), pl.ds(col0(bi), _KC)],
                    in_v.at[s, pl.ds(i * 8, 8), :], semi[s])
                for i in range(4)]

    def out_copy(bi, s):
        return pltpu.make_async_copy(
            out_v.at[s],
            out_hbm.at[pl.ds(pl.multiple_of(col0(bi) // 4, 32), _KC // 4)],
            semo[s])

    def transpose(s):
        @pl.loop(0, _KC // 16, unroll=2)
        def _cgrp(cg):
            c0 = cg * 16
            f0 = jnp.broadcast_to(lax.shift_right_logical(c0, 2), (16,))
            c0v = jnp.broadcast_to(c0, (16,))
            for rg in range(2):
                rowv = rg * 16 + iota16
                o0 = jnp.broadcast_to(rg * 16, (16,))
                for d in range(16):
                    vec = plsc.load_gather(in_v.at[s],
                                           [rowv, c0v + wvec[d]])
                    plsc.store_scatter(out_v.at[s],
                                       [f0 + fvec[d], o0 + ovec[d]], vec)

    @pl.when(blk(0) < lim)
    def _():
        for p in in_pieces(0, 0):
            p.start()

    @pl.when(blk(1) < lim)
    def _():
        for p in in_pieces(1, 1):
            p.start()

    @pl.loop(0, (_BPW + 2) // 2)
    def _pair(t):
        for b in range(2):
            s = b
            bi = 2 * t + b

            @pl.when(blk(bi) < lim)
            def _():
                for p in in_pieces(bi, s):
                    p.wait()

                @pl.when(bi >= 2)
                def _():
                    out_copy(bi - 2, s).wait()

                transpose(s)

                @pl.when(blk(bi + 2) < lim)
                def _():
                    for p in in_pieces(bi + 2, s):
                        p.start()

                out_copy(bi, s).start()

    n_valid = lim - wid * _BPW
    for s in range(2):
        last = ((n_valid - 1 - s) // 2) * 2 + s

        @pl.when(n_valid > s)
        def _():
            out_copy(last, s).wait()


def _gather_body(idx_hbm, table_hbm, out_hbm, idx_v, rows_v, out_t,
                 semi0, semi1, semg0, semg1, semo0, semo1):
    n_rows = idx_hbm.shape[0]
    r_per_w = n_rows // _NW
    n_chunks = r_per_w // _C
    wid = lax.axis_index("s") * _NC + lax.axis_index("c")
    base = wid * r_per_w

    semi = (semi0, semi1)
    semg = (semg0, semg1)
    semo = (semo0, semo1)

    iota16 = lax.iota(jnp.int32, 16)
    wvec = [lax.bitwise_and(d + iota16, 15) for d in range(16)]
    ivec = [lax.shift_right_logical(w, 3) for w in wvec]
    svec = [lax.bitwise_and(w, 7) for w in wvec]

    def idx_copy(c, s):
        return pltpu.make_async_copy(
            idx_hbm.at[pl.ds(base + c * _C, _C)], idx_v.at[s], semi[s])

    def fire_gather(s):
        pltpu.async_copy(table_hbm.at[idx_v.at[s]], rows_v.at[s], semg[s])

    def drain_gather(s):
        pltpu.make_async_copy(table_hbm.at[pl.ds(0, _C)], rows_v.at[s],
                              semg[s]).wait()

    def out_copy(c, s):
        flat0 = base + c * _C
        l = flat0 // _B
        j0 = (flat0 % _B) // 128
        return pltpu.make_async_copy(
            out_t.at[s],
            out_hbm.at[l, :, pl.ds(pl.multiple_of(j0 * 8, 8), 32), :],
            semo[s])

    def out_start(c, s):
        out_copy(c, s).start()

    def out_wait(c, s):
        out_copy(c, s).wait()

    def transpose(s):
        @pl.loop(0, _C // 16, unroll=2)
        def _rgrp(rg):
            r0 = rg * 16
            rowv = r0 + iota16
            lanev = jnp.broadcast_to(lax.bitwise_and(r0, 127), (16,)) + iota16
            mjb = jnp.broadcast_to(lax.shift_right_logical(r0, 7) * 8, (16,))
            for cg in range(2):
                c0v = jnp.broadcast_to(cg * 16, (16,))
                i0 = jnp.broadcast_to(cg * 2, (16,))
                for d in range(16):
                    vec = plsc.load_gather(rows_v.at[s],
                                           [rowv, c0v + wvec[d]])
                    plsc.store_scatter(out_t.at[s],
                                       [i0 + ivec[d], mjb + svec[d], lanev],
                                       vec)

    idx_copy(0, 0).start()
    idx_copy(1, 1).start()
    idx_copy(0, 0).wait()
    fire_gather(0)

    @pl.loop(0, n_chunks // 2)
    def _pair(t):
        for b in range(2):
            c = 2 * t + b
            s = b
            drain_gather(s)

            @pl.when(c + 2 < n_chunks)
            def _():
                idx_copy(c + 2, s).start()

            @pl.when(c + 1 < n_chunks)
            def _():
                idx_copy(c + 1, 1 - s).wait()
                fire_gather(1 - s)

            @pl.when(c >= 2)
            def _():
                out_wait(c - 2, s)

            transpose(s)
            out_start(c, s)

    out_wait(n_chunks - 2, 0)
    out_wait(n_chunks - 1, 1)


def kernel(inp, table):
    b, l = inp.shape
    n = b * l
    idx = inp.T.reshape(n).astype(jnp.int32)
    mesh = plsc.VectorSubcoreMesh(core_axis_name="c", subcore_axis_name="s")

    tbl_rows = pl.kernel(
        _transpose_body,
        out_type=jax.ShapeDtypeStruct((_VP // 4, 128), table.dtype),
        mesh=mesh,
        scratch_types=[
            pltpu.VMEM((2, _DIM, _KC), jnp.float32),
            pltpu.VMEM((2, _KC // 4, 128), jnp.float32),
            pltpu.SemaphoreType.DMA,
            pltpu.SemaphoreType.DMA,
            pltpu.SemaphoreType.DMA,
            pltpu.SemaphoreType.DMA,
        ],
        compiler_params=pltpu.CompilerParams(use_tc_tiling_on_sc=True,
                                             needs_layout_passes=False),
    )(table.T)

    out2 = pl.kernel(
        _gather_body,
        out_type=jax.ShapeDtypeStruct((_L, 4, 256, 128), table.dtype),
        mesh=mesh,
        scratch_types=[
            pltpu.VMEM((2, _C), jnp.int32),
            pltpu.VMEM((2, _C, _DIM), jnp.float32),
            pltpu.VMEM((2, 4, 32, 128), jnp.float32),
            pltpu.SemaphoreType.DMA,
            pltpu.SemaphoreType.DMA,
            pltpu.SemaphoreType.DMA,
            pltpu.SemaphoreType.DMA,
            pltpu.SemaphoreType.DMA,
            pltpu.SemaphoreType.DMA,
        ],
        compiler_params=pltpu.CompilerParams(use_tc_tiling_on_sc=False,
                                             needs_layout_passes=False),
    )(idx, tbl_rows.reshape(_VP, _DIM))

    return (out2.reshape(_L, 4, 32, 8, 128)
                .transpose(2, 4, 0, 1, 3).reshape(b, l, _DIM))
